# baseline (speedup 1.0000x reference)
import jax
import jax.numpy as jnp
from jax.experimental import pallas as pl
from jax.experimental.pallas import tpu as pltpu

_M = 1024
_B = 512
_TS = (0.3 / 3600.0) ** 0.5


_S = 8


def _tile_kernel(dd_ref, sdt_ref, cd_ref, sct_ref, hlp_ref, idsf_ref,
                 asn_ref, w_ref):
    dd = dd_ref[:]
    sdt = sdt_ref[:]

    rmin = jnp.full((_S, _B), jnp.inf, jnp.float32)
    rslabf = jnp.zeros((_S, _B), jnp.float32)
    for s in range(_M // _S):
        cd_c = cd_ref[pl.ds(s * _S, _S), :]
        sct_c = sct_ref[pl.ds(s * _S, _S), :]
        hlp_c = hlp_ref[pl.ds(s * _S, _S), :]
        d1 = dd - cd_c
        t1 = sdt - sct_c
        cost_s = (d1 * d1 + hlp_c) + t1 * t1
        pred = cost_s < rmin
        rmin = jnp.where(pred, cost_s, rmin)
        rslabf = jnp.where(pred, jnp.float32(s), rslabf)

    pat = idsf_ref[pl.ds(0, _S), :]
    jf = rslabf * jnp.float32(_S) + pat
    min_cost = jnp.min(rmin, axis=0, keepdims=True)
    min_jf = jnp.min(jnp.where(rmin == min_cost, jf, 2048.0), axis=0,
                     keepdims=True)
    min_j = min_jf.astype(jnp.int32)

    valid = min_cost < 625.0
    asn_ref[:] = jnp.where(valid, min_j, -1)
    w_ref[:] = jnp.where(valid, 1.0 / (1.0 + jnp.sqrt(min_cost)), 0.0)


def kernel(detection_depths, camera_depths, detection_times, camera_times):
    n = detection_depths.shape[0]
    m = camera_depths.shape[0]
    dd = detection_depths.reshape(1, n)
    sdt = (detection_times * _TS).reshape(1, n)
    cd = camera_depths.reshape(m, 1)
    sct = (camera_times * _TS).reshape(m, 1)
    hlp = (0.5 * (1.0 - jnp.exp(-0.045 * camera_depths))).reshape(m, 1)
    idsf = jnp.arange(m, dtype=jnp.float32).reshape(m, 1)

    grid = (n // _B,)
    asn, w = pl.pallas_call(
        _tile_kernel,
        grid=grid,
        in_specs=[
            pl.BlockSpec((1, _B), lambda i: (0, i)),
            pl.BlockSpec((1, _B), lambda i: (0, i)),
            pl.BlockSpec((m, 1), lambda i: (0, 0)),
            pl.BlockSpec((m, 1), lambda i: (0, 0)),
            pl.BlockSpec((m, 1), lambda i: (0, 0)),
            pl.BlockSpec((m, 1), lambda i: (0, 0)),
        ],
        out_specs=[
            pl.BlockSpec((1, _B), lambda i: (0, i)),
            pl.BlockSpec((1, _B), lambda i: (0, i)),
        ],
        out_shape=[
            jax.ShapeDtypeStruct((1, n), jnp.int32),
            jax.ShapeDtypeStruct((1, n), jnp.float32),
        ],
        compiler_params=pltpu.CompilerParams(
            dimension_semantics=("parallel",)),
    )(dd, sdt, cd, sct, hlp, idsf)

    assignments = asn.reshape(n).astype(jnp.int64)
    weights = w.reshape(n)
    return assignments, weights


# speedup vs baseline: 1.0883x; 1.0883x over previous
import jax
import jax.numpy as jnp
from jax.experimental import pallas as pl
from jax.experimental.pallas import tpu as pltpu

_M = 1024
_B = 1024
_TS = (0.3 / 3600.0) ** 0.5


_S = 8


def _tile_kernel(dd_ref, sdt_ref, cd_ref, sct_ref, hlp_ref, idsf_ref,
                 asn_ref, w_ref):
    dd = dd_ref[:]
    sdt = sdt_ref[:]

    rmin = jnp.full((_S, _B), jnp.inf, jnp.float32)
    rslabf = jnp.zeros((_S, _B), jnp.float32)
    for s in range(_M // _S):
        cd_c = cd_ref[pl.ds(s * _S, _S), :]
        sct_c = sct_ref[pl.ds(s * _S, _S), :]
        hlp_c = hlp_ref[pl.ds(s * _S, _S), :]
        d1 = dd - cd_c
        t1 = sdt - sct_c
        cost_s = (d1 * d1 + hlp_c) + t1 * t1
        pred = cost_s < rmin
        rmin = jnp.where(pred, cost_s, rmin)
        rslabf = jnp.where(pred, jnp.float32(s), rslabf)

    pat = idsf_ref[pl.ds(0, _S), :]
    jf = rslabf * jnp.float32(_S) + pat
    min_cost = jnp.min(rmin, axis=0, keepdims=True)
    min_jf = jnp.min(jnp.where(rmin == min_cost, jf, 2048.0), axis=0,
                     keepdims=True)
    min_j = min_jf.astype(jnp.int32)

    valid = min_cost < 625.0
    asn_ref[:] = jnp.where(valid, min_j, -1)
    w_ref[:] = jnp.where(valid, 1.0 / (1.0 + jnp.sqrt(min_cost)), 0.0)


def kernel(detection_depths, camera_depths, detection_times, camera_times):
    n = detection_depths.shape[0]
    m = camera_depths.shape[0]
    dd = detection_depths.reshape(1, n)
    sdt = (detection_times * _TS).reshape(1, n)
    cd = camera_depths.reshape(m, 1)
    sct = (camera_times * _TS).reshape(m, 1)
    hlp = (0.5 * (1.0 - jnp.exp(-0.045 * camera_depths))).reshape(m, 1)
    idsf = jnp.arange(m, dtype=jnp.float32).reshape(m, 1)

    grid = (n // _B,)
    asn, w = pl.pallas_call(
        _tile_kernel,
        grid=grid,
        in_specs=[
            pl.BlockSpec((1, _B), lambda i: (0, i)),
            pl.BlockSpec((1, _B), lambda i: (0, i)),
            pl.BlockSpec((m, 1), lambda i: (0, 0)),
            pl.BlockSpec((m, 1), lambda i: (0, 0)),
            pl.BlockSpec((m, 1), lambda i: (0, 0)),
            pl.BlockSpec((m, 1), lambda i: (0, 0)),
        ],
        out_specs=[
            pl.BlockSpec((1, _B), lambda i: (0, i)),
            pl.BlockSpec((1, _B), lambda i: (0, i)),
        ],
        out_shape=[
            jax.ShapeDtypeStruct((1, n), jnp.int32),
            jax.ShapeDtypeStruct((1, n), jnp.float32),
        ],
        compiler_params=pltpu.CompilerParams(
            dimension_semantics=("parallel",)),
    )(dd, sdt, cd, sct, hlp, idsf)

    assignments = asn.reshape(n).astype(jnp.int64)
    weights = w.reshape(n)
    return assignments, weights
